# baseline (speedup 1.0000x reference)
import jax
import jax.numpy as jnp
from jax import lax
from jax.experimental import pallas as pl
from jax.experimental.pallas import tpu as pltpu
from jax.experimental.pallas import tpu_sc as plsc

_ANCHORS_W = (116.0, 156.0, 373.0)
_ANCHORS_H = (90.0, 198.0, 326.0)
_IMG_SIZE = 608

_W = 304
_NK = _W // 16


def _sc_body(in_hbm, out_hbm, in_v, out_v):
    G = 76
    S = G * G
    stride = float(_IMG_SIZE // G)
    n_chunks = S // _W
    wid = lax.axis_index("s") * 2 + lax.axis_index("c")
    iota = lax.iota(jnp.int32, 16)
    attrs = 85

    def do_chunk(a, aw, ah, j):
        pltpu.sync_copy(
            in_hbm.at[wid, pl.ds(attrs * a, attrs), pl.ds(j * _W, _W)], in_v)

        def col_body(c, _):
            cols = jnp.full((16,), c, jnp.int32)
            for k in range(_NK):
                t = in_v[c, pl.ds(k * 16, 16)]
                sig = 1.0 / (1.0 + jnp.exp(-t))
                plsc.store_scatter(out_v, [iota + (k * 16), cols], sig)
            return 0

        lax.fori_loop(0, attrs, col_body, 0)

        for c in range(4):
            cols = jnp.full((16,), c, jnp.int32)
            for k in range(_NK):
                t = in_v[c, pl.ds(k * 16, 16)]
                if c == 0:
                    s_vec = iota + (j * _W + k * 16)
                    v = (1.0 / (1.0 + jnp.exp(-t))
                         + (s_vec % G).astype(jnp.float32)) * stride
                elif c == 1:
                    s_vec = iota + (j * _W + k * 16)
                    v = (1.0 / (1.0 + jnp.exp(-t))
                         + (s_vec // G).astype(jnp.float32)) * stride
                elif c == 2:
                    v = jnp.exp(t) * aw
                else:
                    v = jnp.exp(t) * ah
                plsc.store_scatter(out_v, [iota + (k * 16), cols], v)

        pltpu.sync_copy(
            out_v, out_hbm.at[wid, pl.ds(a * S + j * _W, _W), :])
        return j

    for a in range(3):
        lax.fori_loop(
            0, n_chunks,
            lambda j, _, a=a: (do_chunk(a, _ANCHORS_W[a], _ANCHORS_H[a], j), 0)[1],
            0)


def kernel(prediction):
    B, C, G, _ = prediction.shape
    nA = 3
    attrs = C // nA
    S = G * G
    pred2 = prediction.reshape(B, C, S)
    mesh = plsc.VectorSubcoreMesh(core_axis_name="c", subcore_axis_name="s")
    kern = pl.kernel(
        _sc_body,
        out_type=jax.ShapeDtypeStruct((B, nA * S, attrs), jnp.float32),
        mesh=mesh,
        scratch_types=[
            pltpu.VMEM((attrs, _W), jnp.float32),
            pltpu.VMEM((_W, attrs), jnp.float32),
        ],
        compiler_params=pltpu.CompilerParams(use_tc_tiling_on_sc=False, needs_layout_passes=False),
    )
    return kern(pred2)


# speedup vs baseline: 1.4662x; 1.4662x over previous
import jax
import jax.numpy as jnp
from jax import lax
from jax.experimental import pallas as pl
from jax.experimental.pallas import tpu as pltpu
from jax.experimental.pallas import tpu_sc as plsc

_ANCHORS_W = (116.0, 156.0, 373.0)
_ANCHORS_H = (90.0, 198.0, 326.0)
_IMG_SIZE = 608

_W = 304
_NK = _W // 16


def _sc_body(in_hbm, out_hbm, in_v, out_v):
    G = 76
    S = G * G
    stride = float(_IMG_SIZE // G)
    n_chunks = S // _W
    wid = lax.axis_index("s") * 2 + lax.axis_index("c")
    iota = lax.iota(jnp.int32, 16)
    attrs = 85

    def do_chunk(a, aw, ah, j):
        pltpu.sync_copy(
            in_hbm.at[wid, pl.ds(attrs * a, attrs), pl.ds(j * _W, _W)], in_v)

        @plsc.parallel_loop(0, attrs, unroll=2)
        def col_body(c):
            cols = jnp.full((16,), c, jnp.int32)
            for k in range(_NK):
                t = in_v[c, pl.ds(k * 16, 16)]
                sig = 1.0 / (1.0 + jnp.exp(-t))
                plsc.store_scatter(out_v, [iota + (k * 16), cols], sig)

        for c in range(4):
            cols = jnp.full((16,), c, jnp.int32)
            for k in range(_NK):
                t = in_v[c, pl.ds(k * 16, 16)]
                if c == 0:
                    s_vec = iota + (j * _W + k * 16)
                    v = (1.0 / (1.0 + jnp.exp(-t))
                         + (s_vec % G).astype(jnp.float32)) * stride
                elif c == 1:
                    s_vec = iota + (j * _W + k * 16)
                    v = (1.0 / (1.0 + jnp.exp(-t))
                         + (s_vec // G).astype(jnp.float32)) * stride
                elif c == 2:
                    v = jnp.exp(t) * aw
                else:
                    v = jnp.exp(t) * ah
                plsc.store_scatter(out_v, [iota + (k * 16), cols], v)

        pltpu.sync_copy(
            out_v, out_hbm.at[wid, pl.ds(a * S + j * _W, _W), :])
        return j

    for a in range(3):
        lax.fori_loop(
            0, n_chunks,
            lambda j, _, a=a: (do_chunk(a, _ANCHORS_W[a], _ANCHORS_H[a], j), 0)[1],
            0)


def kernel(prediction):
    B, C, G, _ = prediction.shape
    nA = 3
    attrs = C // nA
    S = G * G
    pred2 = prediction.reshape(B, C, S)
    mesh = plsc.VectorSubcoreMesh(core_axis_name="c", subcore_axis_name="s")
    kern = pl.kernel(
        _sc_body,
        out_type=jax.ShapeDtypeStruct((B, nA * S, attrs), jnp.float32),
        mesh=mesh,
        scratch_types=[
            pltpu.VMEM((attrs, _W), jnp.float32),
            pltpu.VMEM((_W, attrs), jnp.float32),
        ],
        compiler_params=pltpu.CompilerParams(use_tc_tiling_on_sc=False, needs_layout_passes=False),
    )
    return kern(pred2)


# speedup vs baseline: 8.7847x; 5.9915x over previous
import functools

import jax
import jax.numpy as jnp
from jax.experimental import pallas as pl

_ANCHORS_W = (116.0, 156.0, 373.0)
_ANCHORS_H = (90.0, 198.0, 326.0)
_IMG_SIZE = 608


def _decode_kernel(in_ref, out_ref, *, G, stride):
    S = G * G
    lane = jax.lax.broadcasted_iota(jnp.int32, (1, S), 1)
    grid_x = (lane % G).astype(jnp.float32)
    grid_y = (lane // G).astype(jnp.float32)
    for b in range(2):
      for a in range(3):
        t = in_ref[b, 85 * a:85 * (a + 1), :]
        sig = jax.nn.sigmoid(t)
        out_ref[b, a] = sig.T
        bx = (sig[0:1] + grid_x) * stride
        by = (sig[1:2] + grid_y) * stride
        bw = jnp.exp(t[2:3]) * _ANCHORS_W[a]
        bh = jnp.exp(t[3:4]) * _ANCHORS_H[a]
        boxes = jnp.concatenate([bx, by, bw, bh], axis=0)
        out_ref[b, a, :, 0:4] = boxes.T


def kernel(prediction):
    B, C, G, _ = prediction.shape
    nA = 3
    attrs = C // nA
    S = G * G
    stride = _IMG_SIZE // G
    pred2 = prediction.reshape(B, C, S)
    out = pl.pallas_call(
        functools.partial(_decode_kernel, G=G, stride=float(stride)),
        grid=(B // 2,),
        in_specs=[pl.BlockSpec((2, C, S), lambda b: (b, 0, 0))],
        out_specs=pl.BlockSpec((2, nA, S, attrs), lambda b: (b, 0, 0, 0)),
        out_shape=jax.ShapeDtypeStruct((B, nA, S, attrs), jnp.float32),
    )(pred2)
    return out.reshape(B, nA * S, attrs)
